# baseline (speedup 1.0000x reference)
import functools

import jax
import jax.numpy as jnp
from jax import lax
from jax.experimental import pallas as pl
from jax.experimental.pallas import tpu as pltpu
from jax.experimental.pallas import tpu_sc as plsc

N_FACTORS = 32
BATCH = 16384
NC = 2
NS = 16
NW = NC * NS
B_PER_W = BATCH // NW
CH = 128
NCHUNK = B_PER_W // CH


def _gather_body(uemb, memb, uidx_hbm, midx_hbm, uout, mout,
                 uidx_v, midx_v, urows_v, mrows_v, usem, msem):
    wid = lax.axis_index("s") * NC + lax.axis_index("c")
    base = wid * B_PER_W
    pltpu.sync_copy(uidx_hbm.at[wid], uidx_v)
    pltpu.sync_copy(midx_hbm.at[wid], midx_v)
    ucopies = []
    mcopies = []
    for j in range(NCHUNK):
        ucopies.append(pltpu.async_copy(
            uemb.at[uidx_v.at[j]], urows_v.at[pl.ds(j * CH, CH)], usem))
        mcopies.append(pltpu.async_copy(
            memb.at[midx_v.at[j]], mrows_v.at[pl.ds(j * CH, CH)], msem))
    for c in ucopies:
        c.wait()
    pltpu.sync_copy(urows_v, uout.at[pl.ds(base, B_PER_W)])
    for c in mcopies:
        c.wait()
    pltpu.sync_copy(mrows_v, mout.at[pl.ds(base, B_PER_W)])


_gather_cache = []


def _gather(*args):
    if not _gather_cache:
        _gather_cache.append(functools.partial(
            pl.kernel,
            mesh=plsc.VectorSubcoreMesh(core_axis_name="c",
                                        subcore_axis_name="s"),
            out_type=[
                jax.ShapeDtypeStruct((BATCH, N_FACTORS), jnp.float32),
                jax.ShapeDtypeStruct((BATCH, N_FACTORS), jnp.float32),
            ],
            scratch_types=[
                pltpu.VMEM((NCHUNK, CH), jnp.int32),
                pltpu.VMEM((NCHUNK, CH), jnp.int32),
                pltpu.VMEM((B_PER_W, N_FACTORS), jnp.float32),
                pltpu.VMEM((B_PER_W, N_FACTORS), jnp.float32),
                pltpu.SemaphoreType.DMA,
                pltpu.SemaphoreType.DMA,
            ],
            compiler_params=pltpu.CompilerParams(use_tc_tiling_on_sc=False),
        )(_gather_body))
    return _gather_cache[0](*args)


def _mlp_body(u_ref, m_ref, w1u_ref, w1m_ref, b1_ref, w2_ref, b2_ref,
              wf_ref, bf_ref, o_ref):
    x = jnp.dot(u_ref[...], w1u_ref[...], preferred_element_type=jnp.float32)
    x = x + jnp.dot(m_ref[...], w1m_ref[...], preferred_element_type=jnp.float32)
    h = jnp.maximum(x + b1_ref[...], 0.0)
    h = jnp.maximum(
        jnp.dot(h, w2_ref[...], preferred_element_type=jnp.float32)
        + b2_ref[...], 0.0)
    s = jnp.dot(h, wf_ref[...], preferred_element_type=jnp.float32) + bf_ref[...]
    o_ref[...] = jax.nn.sigmoid(s) * 4.5 + 0.5


def _mlp(u, m, w1u, w1m, b1, w2, b2, wf, bf, block_b=2048):
    nb = BATCH // block_b
    wspec = lambda shape: pl.BlockSpec(shape, lambda i: (0, 0))
    return pl.pallas_call(
        _mlp_body,
        grid=(nb,),
        in_specs=[
            pl.BlockSpec((block_b, N_FACTORS), lambda i: (i, 0)),
            pl.BlockSpec((block_b, N_FACTORS), lambda i: (i, 0)),
            wspec(w1u.shape),
            wspec(w1m.shape),
            wspec(b1.shape),
            wspec(w2.shape),
            wspec(b2.shape),
            wspec(wf.shape),
            wspec(bf.shape),
        ],
        out_specs=pl.BlockSpec((block_b, 1), lambda i: (i, 0)),
        out_shape=jax.ShapeDtypeStruct((BATCH, 1), jnp.float32),
    )(u, m, w1u, w1m, b1, w2, b2, wf, bf)


@jax.jit
def kernel(users, movies, user_emb, movie_emb, W1, b1, W2, b2, Wf, bf):
    users_r = users.astype(jnp.int32).reshape(NW, NCHUNK, CH)
    movies_r = movies.astype(jnp.int32).reshape(NW, NCHUNK, CH)
    u_rows, m_rows = _gather(user_emb, movie_emb, users_r, movies_r)
    w1u = W1[:N_FACTORS]
    w1m = W1[N_FACTORS:]
    return _mlp(u_rows, m_rows, w1u, w1m,
                b1.reshape(1, -1), W2, b2.reshape(1, -1),
                Wf, bf.reshape(1, 1))
